# baseline (speedup 1.0000x reference)
import functools

import jax
import jax.numpy as jnp
from jax import lax
from jax.experimental import pallas as pl
from jax.experimental.pallas import tpu as pltpu
from jax.experimental.pallas import tpu_sc as plsc

B = 16384
F = 32
L = 16
NC = 2
NS = 16
NW = NC * NS
BPW = B // NW
HB = BPW // 2
RPL = 4
LINES = 1000000 // RPL

_mesh = plsc.VectorSubcoreMesh(core_axis_name="c", subcore_axis_name="s")

_GATHER_DNUMS = lax.GatherDimensionNumbers(
    offset_dims=(), collapsed_slice_dims=(0,), start_index_map=(0,))


def _shuffle(v, idx):
    return lax.gather(v, idx[:, None], _GATHER_DNUMS, (1,),
                      mode=lax.GatherScatterMode.PROMISE_IN_BOUNDS)


@functools.partial(
    pl.kernel,
    mesh=_mesh,
    out_type=jax.ShapeDtypeStruct((B,), jnp.float32),
    scratch_types=[
        pltpu.VMEM((BPW,), jnp.int32),
        pltpu.VMEM((BPW,), jnp.int32),
        pltpu.VMEM((2, 128), jnp.int32),
        pltpu.VMEM((2, 128), jnp.int32),
        pltpu.VMEM((HB, 128), jnp.float32),
        pltpu.VMEM((HB, 128), jnp.float32),
        pltpu.VMEM((BPW,), jnp.float32),
        pltpu.SemaphoreType.DMA,
        pltpu.SemaphoreType.DMA,
    ],
)
def _gmf_kernel(uids_hbm, iids_hbm, utab_hbm, itab_hbm, out_hbm,
                uid_v, iid_v, ul_v, il_v, ur_v, ir_v, o_v, usem, isem):
    wid = lax.axis_index("s") * NC + lax.axis_index("c")
    base = wid * BPW

    pltpu.sync_copy(uids_hbm.at[pl.ds(base, BPW)], uid_v)
    pltpu.sync_copy(iids_hbm.at[pl.ds(base, BPW)], iid_v)

    lanes = lax.iota(jnp.int32, L)
    rolls = [(lanes + d) & (L - 1) for d in (8, 4, 2, 1)]

    def half(h, carry):
        h0 = h * HB

        def lid_body(c, cc):
            for j in range(8):
                off = c * 128 + j * L
                ul_v[c, pl.ds(j * L, L)] = (
                    uid_v[pl.ds(h0 + off, L)] >> 2)
                il_v[c, pl.ds(j * L, L)] = (
                    iid_v[pl.ds(h0 + off, L)] >> 2)
            return cc

        lax.fori_loop(0, 2, lid_body, 0)

        cps = []
        for c in range(2):
            cps.append(pltpu.async_copy(
                utab_hbm.at[ul_v.at[c]],
                ur_v.at[pl.ds(c * 128, 128), :], usem))
            cps.append(pltpu.async_copy(
                itab_hbm.at[il_v.at[c]],
                ir_v.at[pl.ds(c * 128, 128), :], isem))
        for cp in cps:
            cp.wait()

        def group_body(g, cc):
            r0 = g * L
            uvec = uid_v[pl.ds(h0 + r0, L)]
            ivec = iid_v[pl.ds(h0 + r0, L)]
            uq = (uvec & 3) * F
            iq = (ivec & 3) * F
            acc = jnp.zeros((L,), jnp.float32)
            for k in range(L):
                r = r0 + k
                uo = uq[k]
                io = iq[k]
                u0 = ur_v[r, pl.ds(uo, L)]
                u1 = ur_v[r, pl.ds(uo + L, L)]
                i0 = ir_v[r, pl.ds(io, L)]
                i1 = ir_v[r, pl.ds(io + L, L)]
                s = u0 * i0 + u1 * i1
                for rr in rolls:
                    s = s + _shuffle(s, rr)
                acc = jnp.where(lanes == k, s, acc)
            o_v[pl.ds(h0 + r0, L)] = 1.0 / (1.0 + jnp.exp(-acc))
            return cc

        lax.fori_loop(0, HB // L, group_body, 0)
        return carry

    lax.fori_loop(0, 2, half, 0)

    pltpu.sync_copy(o_v, out_hbm.at[pl.ds(base, BPW)])


def kernel(user_ids, item_ids, user_table, item_table):
    return _gmf_kernel(user_ids.astype(jnp.int32), item_ids.astype(jnp.int32),
                       user_table.reshape(LINES, 128),
                       item_table.reshape(LINES, 128))


# speedup vs baseline: 1.0010x; 1.0010x over previous
import functools

import jax
import jax.numpy as jnp
from jax import lax
from jax.experimental import pallas as pl
from jax.experimental.pallas import tpu as pltpu
from jax.experimental.pallas import tpu_sc as plsc

B = 16384
F = 32
L = 16
NC = 2
NS = 16
NW = NC * NS
BPW = B // NW
HB = BPW // 2
RPL = 4
LINES = 1000000 // RPL

_mesh = plsc.VectorSubcoreMesh(core_axis_name="c", subcore_axis_name="s")

_GATHER_DNUMS = lax.GatherDimensionNumbers(
    offset_dims=(), collapsed_slice_dims=(0,), start_index_map=(0,))


def _shuffle(v, idx):
    return lax.gather(v, idx[:, None], _GATHER_DNUMS, (1,),
                      mode=lax.GatherScatterMode.PROMISE_IN_BOUNDS)


@functools.partial(
    pl.kernel,
    mesh=_mesh,
    out_type=jax.ShapeDtypeStruct((B,), jnp.float32),
    scratch_types=[
        pltpu.VMEM((BPW,), jnp.int32),
        pltpu.VMEM((BPW,), jnp.int32),
        pltpu.VMEM((2, 128), jnp.int32),
        pltpu.VMEM((2, 128), jnp.int32),
        pltpu.VMEM((HB, 128), jnp.float32),
        pltpu.VMEM((HB, 128), jnp.float32),
        pltpu.VMEM((BPW,), jnp.float32),
        pltpu.SemaphoreType.DMA,
        pltpu.SemaphoreType.DMA,
    ],
)
def _gmf_kernel(uids_hbm, iids_hbm, utab_hbm, itab_hbm, out_hbm,
                uid_v, iid_v, ul_v, il_v, ur_v, ir_v, o_v, usem, isem):
    wid = lax.axis_index("s") * NC + lax.axis_index("c")
    base = wid * BPW

    pltpu.sync_copy(uids_hbm.at[pl.ds(base, BPW)], uid_v)
    pltpu.sync_copy(iids_hbm.at[pl.ds(base, BPW)], iid_v)

    lanes = lax.iota(jnp.int32, L)
    rolls = [(lanes + d) & (L - 1) for d in (8, 4, 2, 1)]

    def half(h, carry):
        h0 = h * HB

        def lid_body(c, cc):
            for j in range(8):
                off = c * 128 + j * L
                ul_v[c, pl.ds(j * L, L)] = (
                    uid_v[pl.ds(h0 + off, L)] >> 2)
                il_v[c, pl.ds(j * L, L)] = (
                    iid_v[pl.ds(h0 + off, L)] >> 2)
            return cc

        lax.fori_loop(0, 2, lid_body, 0)

        cps = []
        for c in range(2):
            cps.append(pltpu.async_copy(
                utab_hbm.at[ul_v.at[c]],
                ur_v.at[pl.ds(c * 128, 128), :], usem))
            cps.append(pltpu.async_copy(
                itab_hbm.at[il_v.at[c]],
                ir_v.at[pl.ds(c * 128, 128), :], isem))
        for cp in cps:
            cp.wait()

        def group_body(g, cc):
            r0 = g * L
            uvec = uid_v[pl.ds(h0 + r0, L)]
            ivec = iid_v[pl.ds(h0 + r0, L)]
            uq = (uvec & 3) * F
            iq = (ivec & 3) * F
            acc = jnp.zeros((L,), jnp.float32)
            for k in range(L):
                r = r0 + k
                uo = uq[k]
                io = iq[k]
                u0 = ur_v[r, pl.ds(uo, L)]
                u1 = ur_v[r, pl.ds(uo + L, L)]
                i0 = ir_v[r, pl.ds(io, L)]
                i1 = ir_v[r, pl.ds(io + L, L)]
                s = u0 * i0 + u1 * i1
                for rr in rolls:
                    s = s + _shuffle(s, rr)
                acc = jnp.where(lanes == k, s, acc)
            o_v[pl.ds(h0 + r0, L)] = 1.0 / (1.0 + jnp.exp(-acc))
            return cc

        lax.fori_loop(0, HB // L, group_body, 0)
        return carry

    lax.fori_loop(0, 2, half, 0)

    pltpu.sync_copy(o_v, out_hbm.at[pl.ds(base, BPW)])


def kernel(user_ids, item_ids, user_table, item_table):
    ut = (user_table + 0.0).reshape(LINES, 128)
    it = (item_table + 0.0).reshape(LINES, 128)
    return _gmf_kernel(user_ids.astype(jnp.int32), item_ids.astype(jnp.int32),
                       ut, it)


# speedup vs baseline: 1.1033x; 1.1022x over previous
import functools

import jax
import jax.numpy as jnp
from jax import lax
from jax.experimental import pallas as pl
from jax.experimental.pallas import tpu as pltpu
from jax.experimental.pallas import tpu_sc as plsc

B = 16384
F = 32
L = 16
NC = 2
NS = 16
NW = NC * NS
BPW = B // NW
HB = BPW // 2
RPL = 4
LINES = 1000000 // RPL

_mesh = plsc.VectorSubcoreMesh(core_axis_name="c", subcore_axis_name="s")

_GATHER_DNUMS = lax.GatherDimensionNumbers(
    offset_dims=(), collapsed_slice_dims=(0,), start_index_map=(0,))


def _shuffle(v, idx):
    return lax.gather(v, idx[:, None], _GATHER_DNUMS, (1,),
                      mode=lax.GatherScatterMode.PROMISE_IN_BOUNDS)


@functools.partial(
    pl.kernel,
    mesh=_mesh,
    out_type=jax.ShapeDtypeStruct((B,), jnp.float32),
    scratch_types=[
        pltpu.VMEM((BPW,), jnp.int32),
        pltpu.VMEM((BPW,), jnp.int32),
        pltpu.VMEM((2, 128), jnp.int32),
        pltpu.VMEM((2, 128), jnp.int32),
        pltpu.VMEM((HB, 128), jnp.float32),
        pltpu.VMEM((HB, 128), jnp.float32),
        pltpu.VMEM((BPW,), jnp.float32),
        pltpu.SemaphoreType.DMA,
        pltpu.SemaphoreType.DMA,
    ],
)
def _gmf_kernel(uids_hbm, iids_hbm, utab_hbm, itab_hbm, out_hbm,
                uid_v, iid_v, ul_v, il_v, ur_v, ir_v, o_v, usem, isem):
    wid = lax.axis_index("s") * NC + lax.axis_index("c")
    base = wid * BPW

    pltpu.sync_copy(uids_hbm.at[pl.ds(base, BPW)], uid_v)
    pltpu.sync_copy(iids_hbm.at[pl.ds(base, BPW)], iid_v)

    lanes = lax.iota(jnp.int32, L)
    rolls = [(lanes + d) & (L - 1) for d in (8, 4, 2, 1)]

    def half(h, carry):
        h0 = h * HB

        def lid_body(c, cc):
            for j in range(8):
                off = c * 128 + j * L
                ul_v[c, pl.ds(j * L, L)] = (
                    uid_v[pl.ds(h0 + off, L)] >> 2)
                il_v[c, pl.ds(j * L, L)] = (
                    iid_v[pl.ds(h0 + off, L)] >> 2)
            return cc

        lax.fori_loop(0, 2, lid_body, 0)

        cps = []
        for c in range(2):
            cps.append(pltpu.async_copy(
                utab_hbm.at[ul_v.at[c]],
                ur_v.at[pl.ds(c * 128, 128), :], usem))
            cps.append(pltpu.async_copy(
                itab_hbm.at[il_v.at[c]],
                ir_v.at[pl.ds(c * 128, 128), :], isem))
        for cp in cps:
            cp.wait()

        def group_body(g, cc):
            r0 = g * L
            uvec = uid_v[pl.ds(h0 + r0, L)]
            ivec = iid_v[pl.ds(h0 + r0, L)]
            uq = (uvec & 3) * F
            iq = (ivec & 3) * F
            acc = jnp.zeros((L,), jnp.float32)
            for k in range(L):
                r = r0 + k
                uo = uq[k]
                io = iq[k]
                u0 = ur_v[r, pl.ds(uo, L)]
                u1 = ur_v[r, pl.ds(uo + L, L)]
                i0 = ir_v[r, pl.ds(io, L)]
                i1 = ir_v[r, pl.ds(io + L, L)]
                s = u0 * i0 + u1 * i1
                for rr in rolls:
                    s = s + _shuffle(s, rr)
                acc = jnp.where(lanes == k, s, acc)
            o_v[pl.ds(h0 + r0, L)] = 1.0 / (1.0 + jnp.exp(-acc))
            return cc

        lax.fori_loop(0, HB // L, group_body, 0)
        return carry

    lax.fori_loop(0, 2, half, 0)

    pltpu.sync_copy(o_v, out_hbm.at[pl.ds(base, BPW)])


_UB = 4096
_GRID = -(-1000000 // _UB)


def _relayout_body(in_ref, out_ref):
    y = jnp.swapaxes(in_ref[...], 0, 1)
    y3 = y.reshape(_UB // RPL, RPL, F)
    out_ref[...] = jnp.concatenate([y3[:, q, :] for q in range(RPL)], axis=1)


def _to_lines(table_t):
    return pl.pallas_call(
        _relayout_body,
        grid=(_GRID,),
        in_specs=[pl.BlockSpec((F, _UB), lambda b: (0, b))],
        out_specs=pl.BlockSpec((_UB // RPL, 128), lambda b: (b, 0)),
        out_shape=jax.ShapeDtypeStruct((LINES, 128), jnp.float32),
    )(table_t)


def kernel(user_ids, item_ids, user_table, item_table):
    return _gmf_kernel(user_ids.astype(jnp.int32), item_ids.astype(jnp.int32),
                       _to_lines(user_table.T), _to_lines(item_table.T))


# speedup vs baseline: 1.1754x; 1.0653x over previous
import functools

import jax
import jax.numpy as jnp
from jax import lax
from jax.experimental import pallas as pl
from jax.experimental.pallas import tpu as pltpu
from jax.experimental.pallas import tpu_sc as plsc

B = 16384
F = 32
L = 16
NC = 2
NS = 16
NW = NC * NS
BPW = B // NW
HB = BPW // 2
RPL = 4
LINES = 1000000 // RPL

_mesh = plsc.VectorSubcoreMesh(core_axis_name="c", subcore_axis_name="s")

_GATHER_DNUMS = lax.GatherDimensionNumbers(
    offset_dims=(), collapsed_slice_dims=(0,), start_index_map=(0,))


def _shuffle(v, idx):
    return lax.gather(v, idx[:, None], _GATHER_DNUMS, (1,),
                      mode=lax.GatherScatterMode.PROMISE_IN_BOUNDS)


@functools.partial(
    pl.kernel,
    mesh=_mesh,
    out_type=jax.ShapeDtypeStruct((B,), jnp.float32),
    scratch_types=[
        pltpu.VMEM((BPW,), jnp.int32),
        pltpu.VMEM((BPW,), jnp.int32),
        pltpu.VMEM((2, 128), jnp.int32),
        pltpu.VMEM((2, 128), jnp.int32),
        pltpu.VMEM((HB, 128), jnp.float32),
        pltpu.VMEM((HB, 128), jnp.float32),
        pltpu.VMEM((BPW,), jnp.float32),
        pltpu.SemaphoreType.DMA,
        pltpu.SemaphoreType.DMA,
    ],
)
def _gmf_kernel(uids_hbm, iids_hbm, utab_hbm, itab_hbm, out_hbm,
                uid_v, iid_v, ul_v, il_v, ur_v, ir_v, o_v, usem, isem):
    wid = lax.axis_index("s") * NC + lax.axis_index("c")
    base = wid * BPW

    pltpu.sync_copy(uids_hbm.at[pl.ds(base, BPW)], uid_v)
    pltpu.sync_copy(iids_hbm.at[pl.ds(base, BPW)], iid_v)

    lanes = lax.iota(jnp.int32, L)
    rolls = [(lanes + d) & (L - 1) for d in (8, 4, 2, 1)]

    def half(h, carry):
        h0 = h * HB

        def lid_body(c, cc):
            for j in range(8):
                off = c * 128 + j * L
                ul_v[c, pl.ds(j * L, L)] = (
                    uid_v[pl.ds(h0 + off, L)] >> 2)
                il_v[c, pl.ds(j * L, L)] = (
                    iid_v[pl.ds(h0 + off, L)] >> 2)
            return cc

        lax.fori_loop(0, 2, lid_body, 0)

        cps = []
        for c in range(2):
            cps.append(pltpu.async_copy(
                utab_hbm.at[ul_v.at[c]],
                ur_v.at[pl.ds(c * 128, 128), :], usem))
            cps.append(pltpu.async_copy(
                itab_hbm.at[il_v.at[c]],
                ir_v.at[pl.ds(c * 128, 128), :], isem))
        for cp in cps:
            cp.wait()

        def group_body(g, cc):
            r0 = g * L
            uvec = uid_v[pl.ds(h0 + r0, L)]
            ivec = iid_v[pl.ds(h0 + r0, L)]
            uq = (uvec & 3) * F
            iq = (ivec & 3) * F
            acc = jnp.zeros((L,), jnp.float32)
            for k in range(L):
                r = r0 + k
                uo = uq[k]
                io = iq[k]
                u0 = ur_v[r, pl.ds(uo, L)]
                u1 = ur_v[r, pl.ds(uo + L, L)]
                i0 = ir_v[r, pl.ds(io, L)]
                i1 = ir_v[r, pl.ds(io + L, L)]
                s = u0 * i0 + u1 * i1
                for rr in rolls:
                    s = s + _shuffle(s, rr)
                acc = jnp.where(lanes == k, s, acc)
            o_v[pl.ds(h0 + r0, L)] = 1.0 / (1.0 + jnp.exp(-acc))
            return cc

        lax.fori_loop(0, HB // L, group_body, 0)
        return carry

    lax.fori_loop(0, 2, half, 0)

    pltpu.sync_copy(o_v, out_hbm.at[pl.ds(base, BPW)])


_UB = 4096
_GRID = -(-1000000 // _UB)


def _relayout_body(in_ref, out_ref):
    y = jnp.swapaxes(in_ref[...], 0, 1)
    y3 = y.reshape(_UB // RPL, RPL, F)
    out_ref[...] = jnp.concatenate([y3[:, q, :] for q in range(RPL)], axis=1)


def _to_lines(table_t):
    return pl.pallas_call(
        _relayout_body,
        grid=(_GRID,),
        in_specs=[pl.BlockSpec((F, _UB), lambda b: (0, b))],
        out_specs=pl.BlockSpec((_UB // RPL, 128), lambda b: (b, 0)),
        out_shape=jax.ShapeDtypeStruct((LINES, 128), jnp.float32),
    )(table_t)


def kernel(user_ids, item_ids, user_table, item_table):
    return _gmf_kernel(user_ids.astype(jnp.int32), item_ids.astype(jnp.int32),
                       user_table.reshape(LINES, 128),
                       _to_lines(item_table.T))


# speedup vs baseline: 1.2034x; 1.0238x over previous
import functools

import jax
import jax.numpy as jnp
from jax import lax
from jax.experimental import pallas as pl
from jax.experimental.pallas import tpu as pltpu
from jax.experimental.pallas import tpu_sc as plsc

B = 16384
F = 32
L = 16
NC = 2
NS = 16
NW = NC * NS
BPW = B // NW
HB = BPW // 2
RPL = 4
LINES = 1000000 // RPL

_mesh = plsc.VectorSubcoreMesh(core_axis_name="c", subcore_axis_name="s")

_GATHER_DNUMS = lax.GatherDimensionNumbers(
    offset_dims=(), collapsed_slice_dims=(0,), start_index_map=(0,))


def _shuffle(v, idx):
    return lax.gather(v, idx[:, None], _GATHER_DNUMS, (1,),
                      mode=lax.GatherScatterMode.PROMISE_IN_BOUNDS)


@functools.partial(
    pl.kernel,
    mesh=_mesh,
    out_type=jax.ShapeDtypeStruct((B,), jnp.float32),
    scratch_types=[
        pltpu.VMEM((BPW,), jnp.int32),
        pltpu.VMEM((BPW,), jnp.int32),
        pltpu.VMEM((2, 128), jnp.int32),
        pltpu.VMEM((2, 128), jnp.int32),
        pltpu.VMEM((HB, 128), jnp.float32),
        pltpu.VMEM((HB, 128), jnp.float32),
        pltpu.VMEM((BPW,), jnp.float32),
        pltpu.SemaphoreType.DMA,
        pltpu.SemaphoreType.DMA,
    ],
)
def _gmf_kernel(uids_hbm, iids_hbm, utab_hbm, itab_hbm, out_hbm,
                uid_v, iid_v, ul_v, il_v, ur_v, ir_v, o_v, usem, isem):
    wid = lax.axis_index("s") * NC + lax.axis_index("c")
    base = wid * BPW

    pltpu.sync_copy(uids_hbm.at[pl.ds(base, BPW)], uid_v)
    pltpu.sync_copy(iids_hbm.at[pl.ds(base, BPW)], iid_v)

    lanes = lax.iota(jnp.int32, L)
    rolls = [(lanes + d) & (L - 1) for d in (8, 4, 2, 1)]

    def half(h, carry):
        h0 = h * HB

        def lid_body(c, cc):
            for j in range(8):
                off = c * 128 + j * L
                ul_v[c, pl.ds(j * L, L)] = (
                    uid_v[pl.ds(h0 + off, L)] >> 2)
                il_v[c, pl.ds(j * L, L)] = (
                    iid_v[pl.ds(h0 + off, L)] >> 2)
            return cc

        lax.fori_loop(0, 2, lid_body, 0)

        cps = []
        for c in range(2):
            cps.append(pltpu.async_copy(
                utab_hbm.at[ul_v.at[c]],
                ur_v.at[pl.ds(c * 128, 128), :], usem))
            cps.append(pltpu.async_copy(
                itab_hbm.at[il_v.at[c]],
                ir_v.at[pl.ds(c * 128, 128), :], isem))
        for cp in cps:
            cp.wait()

        def group_body(g, cc):
            r0 = g * L
            uvec = uid_v[pl.ds(h0 + r0, L)]
            ivec = iid_v[pl.ds(h0 + r0, L)]
            uq = (uvec & 3) * F
            iq = (ivec & 3) * F
            acc = jnp.zeros((L,), jnp.float32)
            for k in range(L):
                r = r0 + k
                uo = uq[k]
                io = iq[k]
                u0 = ur_v[r, pl.ds(uo, L)]
                u1 = ur_v[r, pl.ds(uo + L, L)]
                i0 = ir_v[r, pl.ds(io, L)]
                i1 = ir_v[r, pl.ds(io + L, L)]
                s = u0 * i0 + u1 * i1
                for rr in rolls:
                    s = s + _shuffle(s, rr)
                acc = jnp.where(lanes == k, s, acc)
            o_v[pl.ds(h0 + r0, L)] = 1.0 / (1.0 + jnp.exp(-acc))
            return cc

        lax.fori_loop(0, HB // L, group_body, 0)
        return carry

    lax.fori_loop(0, 2, half, 0)

    pltpu.sync_copy(o_v, out_hbm.at[pl.ds(base, BPW)])


_UB = 8192
_GRID = -(-1000000 // _UB)


def _relayout_body(in_ref, out_ref):
    y = jnp.swapaxes(in_ref[...], 0, 1)
    y3 = y.reshape(_UB // RPL, RPL, F)
    out_ref[...] = jnp.concatenate([y3[:, q, :] for q in range(RPL)], axis=1)


def _to_lines(table_t):
    return pl.pallas_call(
        _relayout_body,
        grid=(_GRID,),
        in_specs=[pl.BlockSpec((F, _UB), lambda b: (0, b))],
        out_specs=pl.BlockSpec((_UB // RPL, 128), lambda b: (b, 0)),
        out_shape=jax.ShapeDtypeStruct((LINES, 128), jnp.float32),
    )(table_t)


def kernel(user_ids, item_ids, user_table, item_table):
    return _gmf_kernel(user_ids.astype(jnp.int32), item_ids.astype(jnp.int32),
                       user_table.reshape(LINES, 128),
                       _to_lines(item_table.T))


# speedup vs baseline: 1.2155x; 1.0100x over previous
import functools

import jax
import jax.numpy as jnp
from jax import lax
from jax.experimental import pallas as pl
from jax.experimental.pallas import tpu as pltpu
from jax.experimental.pallas import tpu_sc as plsc

B = 16384
F = 32
L = 16
NC = 2
NS = 16
NW = NC * NS
BPW = B // NW
HB = BPW // 2
RPL = 4
LINES = 1000000 // RPL

_mesh = plsc.VectorSubcoreMesh(core_axis_name="c", subcore_axis_name="s")

_GATHER_DNUMS = lax.GatherDimensionNumbers(
    offset_dims=(), collapsed_slice_dims=(0,), start_index_map=(0,))


def _shuffle(v, idx):
    return lax.gather(v, idx[:, None], _GATHER_DNUMS, (1,),
                      mode=lax.GatherScatterMode.PROMISE_IN_BOUNDS)


@functools.partial(
    pl.kernel,
    mesh=_mesh,
    out_type=jax.ShapeDtypeStruct((B,), jnp.float32),
    scratch_types=[
        pltpu.VMEM((BPW,), jnp.int32),
        pltpu.VMEM((BPW,), jnp.int32),
        pltpu.VMEM((2, 128), jnp.int32),
        pltpu.VMEM((2, 128), jnp.int32),
        pltpu.VMEM((HB, 128), jnp.float32),
        pltpu.VMEM((HB, 128), jnp.float32),
        pltpu.VMEM((BPW,), jnp.float32),
        pltpu.SemaphoreType.DMA,
        pltpu.SemaphoreType.DMA,
    ],
)
def _gmf_kernel(uids_hbm, iids_hbm, utab_hbm, itab_hbm, out_hbm,
                uid_v, iid_v, ul_v, il_v, ur_v, ir_v, o_v, usem, isem):
    wid = lax.axis_index("s") * NC + lax.axis_index("c")
    base = wid * BPW

    pltpu.sync_copy(uids_hbm.at[pl.ds(base, BPW)], uid_v)
    pltpu.sync_copy(iids_hbm.at[pl.ds(base, BPW)], iid_v)

    lanes = lax.iota(jnp.int32, L)
    rolls = [(lanes + d) & (L - 1) for d in (8, 4, 2, 1)]

    def half(h, carry):
        h0 = h * HB

        def lid_body(c, cc):
            for j in range(8):
                off = c * 128 + j * L
                ul_v[c, pl.ds(j * L, L)] = (
                    uid_v[pl.ds(h0 + off, L)] >> 2)
                il_v[c, pl.ds(j * L, L)] = (
                    iid_v[pl.ds(h0 + off, L)] >> 2)
            return cc

        lax.fori_loop(0, 2, lid_body, 0)

        cps = []
        for c in range(2):
            cps.append(pltpu.async_copy(
                utab_hbm.at[ul_v.at[c]],
                ur_v.at[pl.ds(c * 128, 128), :], usem))
            cps.append(pltpu.async_copy(
                itab_hbm.at[il_v.at[c]],
                ir_v.at[pl.ds(c * 128, 128), :], isem))
        for cp in cps:
            cp.wait()

        def group_body(g, cc):
            r0 = g * L
            uvec = uid_v[pl.ds(h0 + r0, L)]
            ivec = iid_v[pl.ds(h0 + r0, L)]
            uq = (uvec & 3) * F
            iq = (ivec & 3) * F
            acc = jnp.zeros((L,), jnp.float32)
            for k in range(L):
                r = r0 + k
                uo = uq[k]
                io = iq[k]
                u0 = ur_v[r, pl.ds(uo, L)]
                u1 = ur_v[r, pl.ds(uo + L, L)]
                i0 = ir_v[r, pl.ds(io, L)]
                i1 = ir_v[r, pl.ds(io + L, L)]
                s = u0 * i0 + u1 * i1
                for rr in rolls:
                    s = s + _shuffle(s, rr)
                acc = jnp.where(lanes == k, s, acc)
            o_v[pl.ds(h0 + r0, L)] = 1.0 / (1.0 + jnp.exp(-acc))
            return cc

        lax.fori_loop(0, HB // L, group_body, 0)
        return carry

    lax.fori_loop(0, 2, half, 0)

    pltpu.sync_copy(o_v, out_hbm.at[pl.ds(base, BPW)])


_UB = 16384
_GRID = -(-1000000 // _UB)


def _relayout_body(in_ref, out_ref):
    y = jnp.swapaxes(in_ref[...], 0, 1)
    y3 = y.reshape(_UB // RPL, RPL, F)
    out_ref[...] = jnp.concatenate([y3[:, q, :] for q in range(RPL)], axis=1)


def _to_lines(table_t):
    return pl.pallas_call(
        _relayout_body,
        grid=(_GRID,),
        in_specs=[pl.BlockSpec((F, _UB), lambda b: (0, b))],
        out_specs=pl.BlockSpec((_UB // RPL, 128), lambda b: (b, 0)),
        out_shape=jax.ShapeDtypeStruct((LINES, 128), jnp.float32),
    )(table_t)


def kernel(user_ids, item_ids, user_table, item_table):
    return _gmf_kernel(user_ids.astype(jnp.int32), item_ids.astype(jnp.int32),
                       user_table.reshape(LINES, 128),
                       _to_lines(item_table.T))
